# baseline (speedup 1.0000x reference)
import dataclasses

import jax
import jax.numpy as jnp
from jax.experimental import pallas as pl
from jax.experimental.pallas import tpu as pltpu
from jax.experimental.pallas import tpu_sc as plsc


def _sc_compiler_params():
    cp = pltpu.CompilerParams()
    if "needs_layout_passes" in pltpu.CompilerParams.__dataclass_fields__:
        cp = dataclasses.replace(cp, needs_layout_passes=False)
    return cp


def _split_hi_lo(v):
    hi = v.astype(jnp.bfloat16)
    lo = (v - hi.astype(jnp.float32)).astype(jnp.bfloat16)
    return hi, lo


def _build_s_sparsecore(idxt, n_units, n_rec):
    n_k = idxt.shape[0]
    cols_per_tec = n_rec // 32
    tile_w = 128
    n_chunks = cols_per_tec // tile_w

    mesh = plsc.VectorSubcoreMesh(core_axis_name="c", subcore_axis_name="s")

    @pl.kernel(
        out_type=jax.ShapeDtypeStruct((n_units, n_rec), jnp.float32),
        mesh=mesh,
        scratch_types=[
            pltpu.VMEM((n_units, tile_w), jnp.float32),
            pltpu.VMEM((n_k, cols_per_tec), jnp.int32),
            pltpu.SemaphoreType.DMA,
        ],
        compiler_params=_sc_compiler_params(),
    )
    def build(idx_hbm, s_hbm, tile, idx_vmem, sem):
        sc_id = jax.lax.axis_index("c")
        tec_id = jax.lax.axis_index("s")
        base = (sc_id * 16 + tec_id) * cols_per_tec
        pltpu.async_copy(idx_hbm.at[:, pl.ds(base, cols_per_tec)], idx_vmem, sem).wait()
        zeros16 = jnp.zeros((16,), jnp.float32)
        ones16 = jnp.ones((16,), jnp.float32)
        lane0 = jax.lax.iota(jnp.int32, 16)
        for chunk in range(n_chunks):
            @pl.loop(0, n_units)
            def _(row):
                for g in range(tile_w // 16):
                    tile[row, pl.ds(g * 16, 16)] = zeros16
            for g in range(tile_w // 16):
                lanes = lane0 + g * 16
                for k in range(n_k):
                    rows = idx_vmem[k, pl.ds(chunk * tile_w + g * 16, 16)]
                    plsc.addupdate_scatter(tile, [rows, lanes], ones16)
            pltpu.async_copy(
                tile, s_hbm.at[:, pl.ds(base + chunk * tile_w, tile_w)], sem
            ).wait()

    return build(idxt)


def _mwc_kernel(
    eo_ref, ec_ref, c_ref, s_ref, eps_ref, out_ref,
    ph_scr, plo_scr, dh_scr, dlo_scr,
):
    ir = pl.program_id(0)
    n_units = eo_ref.shape[1]
    bb = out_ref.shape[0]

    @pl.when(ir == 0)
    def _():
        c = c_ref[...]
        eo = eo_ref[...]
        ec = ec_ref[...]
        p = jnp.log1p(c * jnp.exp(-ec)) - jnp.log1p(c * jnp.exp(-eo))
        ph, plo = _split_hi_lo(p)
        eh, elo = _split_hi_lo(eps_ref[...])
        zpad = jnp.zeros((7, n_units), jnp.bfloat16)
        ph_scr[...] = jnp.concatenate([ph, eh, zpad], axis=0)
        plo_scr[...] = jnp.concatenate([plo, elo, zpad], axis=0)
        dh_scr[...], dlo_scr[...] = _split_hi_lo(eo - ec)

    sb = s_ref[...].astype(jnp.bfloat16)

    xf = jnp.dot(ph_scr[...], sb, preferred_element_type=jnp.float32) + jnp.dot(
        plo_scr[...], sb, preferred_element_type=jnp.float32
    )
    x = xf[:bb, :]
    er = xf[bb : bb + 1, :]
    sd = jnp.dot(dh_scr[...], sb, preferred_element_type=jnp.float32) + jnp.dot(
        dlo_scr[...], sb, preferred_element_type=jnp.float32
    )

    L = jnp.exp(-er)
    p_min = 1.0 / (1.0 + L)
    p_c = 1.0 / (1.0 + L * jnp.exp(x))
    p_max = 1.0 / (1.0 + L * jnp.exp(sd))
    denom = p_max - p_min
    norm = (p_c - p_min) / (denom + 1e-8)
    norm = jnp.where(denom > 1e-6, norm, 0.0)
    out_ref[...] = jnp.clip(norm, 0.0, 1.0)


@jax.jit
def kernel(energies, concentrations, receptor_indices, epsilon_units):
    b, u, _ = energies.shape
    r, k = receptor_indices.shape
    br = 512
    nr = r // br

    idxt = receptor_indices.T
    s = _build_s_sparsecore(idxt, u, r)

    one = jnp.float32(1.0)
    eo = energies[:, :, 0] * one
    ec = energies[:, :, 1] * one
    c2 = concentrations.reshape(b, 1)
    eps2 = epsilon_units.reshape(1, u)

    return pl.pallas_call(
        _mwc_kernel,
        grid=(nr,),
        in_specs=[
            pl.BlockSpec((b, u), lambda ir: (0, 0)),
            pl.BlockSpec((b, u), lambda ir: (0, 0)),
            pl.BlockSpec((b, 1), lambda ir: (0, 0)),
            pl.BlockSpec((u, br), lambda ir: (0, ir)),
            pl.BlockSpec((1, u), lambda ir: (0, 0)),
        ],
        out_specs=pl.BlockSpec((b, br), lambda ir: (0, ir)),
        out_shape=jax.ShapeDtypeStruct((b, r), jnp.float32),
        scratch_shapes=[
            pltpu.VMEM((b + 8, u), jnp.bfloat16),
            pltpu.VMEM((b + 8, u), jnp.bfloat16),
            pltpu.VMEM((b, u), jnp.bfloat16),
            pltpu.VMEM((b, u), jnp.bfloat16),
        ],
    )(eo, ec, c2, s, eps2)


# speedup vs baseline: 1.1858x; 1.1858x over previous
import jax
import jax.numpy as jnp
from jax.experimental import pallas as pl
from jax.experimental.pallas import tpu as pltpu


def _split_hi_lo(v):
    hi = v.astype(jnp.bfloat16)
    lo = (v - hi.astype(jnp.float32)).astype(jnp.bfloat16)
    return hi, lo


def _mwc_kernel(
    eo_ref, ec_ref, c_ref, idx_ref, eps_ref, out_ref,
    ph_scr, plo_scr, dh_scr, dlo_scr,
):
    ir = pl.program_id(0)
    n_units = eo_ref.shape[1]
    bb = out_ref.shape[0]
    br = out_ref.shape[1]

    @pl.when(ir == 0)
    def _():
        c = c_ref[...]
        eo = eo_ref[...]
        ec = ec_ref[...]
        p = jnp.log1p(c * jnp.exp(-ec)) - jnp.log1p(c * jnp.exp(-eo))
        ph, plo = _split_hi_lo(p)
        eh, elo = _split_hi_lo(eps_ref[...])
        zpad = jnp.zeros((7, n_units), jnp.bfloat16)
        ph_scr[...] = jnp.concatenate([ph, eh, zpad], axis=0)
        plo_scr[...] = jnp.concatenate([plo, elo, zpad], axis=0)
        dh_scr[...], dlo_scr[...] = _split_hi_lo(eo - ec)

    idx = idx_ref[...]
    u_iota = jax.lax.broadcasted_iota(jnp.int32, (n_units, br), 0)
    s = jnp.zeros((n_units, br), jnp.float32)
    for k in range(idx_ref.shape[0]):
        s = s + jnp.where(u_iota == idx[k : k + 1, :], 1.0, 0.0)
    sb = s.astype(jnp.bfloat16)

    xf = jnp.dot(ph_scr[...], sb, preferred_element_type=jnp.float32) + jnp.dot(
        plo_scr[...], sb, preferred_element_type=jnp.float32
    )
    x = xf[:bb, :]
    er = xf[bb : bb + 1, :]
    sd = jnp.dot(dh_scr[...], sb, preferred_element_type=jnp.float32) + jnp.dot(
        dlo_scr[...], sb, preferred_element_type=jnp.float32
    )

    L = jnp.exp(-er)
    p_min = 1.0 / (1.0 + L)
    p_c = 1.0 / (1.0 + L * jnp.exp(x))
    p_max = 1.0 / (1.0 + L * jnp.exp(sd))
    denom = p_max - p_min
    norm = (p_c - p_min) / (denom + 1e-8)
    norm = jnp.where(denom > 1e-6, norm, 0.0)
    out_ref[...] = jnp.clip(norm, 0.0, 1.0)


@jax.jit
def kernel(energies, concentrations, receptor_indices, epsilon_units):
    b, u, _ = energies.shape
    r, k = receptor_indices.shape
    br = 512
    nr = r // br

    one = jnp.float32(1.0)
    eo = energies[:, :, 0] * one
    ec = energies[:, :, 1] * one
    c2 = concentrations.reshape(b, 1)
    idxt = receptor_indices.T
    eps2 = epsilon_units.reshape(1, u)

    return pl.pallas_call(
        _mwc_kernel,
        grid=(nr,),
        in_specs=[
            pl.BlockSpec((b, u), lambda ir: (0, 0)),
            pl.BlockSpec((b, u), lambda ir: (0, 0)),
            pl.BlockSpec((b, 1), lambda ir: (0, 0)),
            pl.BlockSpec((k, br), lambda ir: (0, ir)),
            pl.BlockSpec((1, u), lambda ir: (0, 0)),
        ],
        out_specs=pl.BlockSpec((b, br), lambda ir: (0, ir)),
        out_shape=jax.ShapeDtypeStruct((b, r), jnp.float32),
        scratch_shapes=[
            pltpu.VMEM((b + 8, u), jnp.bfloat16),
            pltpu.VMEM((b + 8, u), jnp.bfloat16),
            pltpu.VMEM((b, u), jnp.bfloat16),
            pltpu.VMEM((b, u), jnp.bfloat16),
        ],
    )(eo, ec, c2, idxt, eps2)


# speedup vs baseline: 1.2471x; 1.0517x over previous
import jax
import jax.numpy as jnp
from jax.experimental import pallas as pl
from jax.experimental.pallas import tpu as pltpu


def _split_hi_lo(v):
    hi = v.astype(jnp.bfloat16)
    lo = (v - hi.astype(jnp.float32)).astype(jnp.bfloat16)
    return hi, lo


def _mwc_kernel(
    eo_ref, ec_ref, c_ref, idx_ref, eps_ref, out_ref,
    ph_scr, plo_scr, dh_scr, dlo_scr,
):
    ir = pl.program_id(0)
    n_units = eo_ref.shape[1]
    bb = out_ref.shape[0]
    br = out_ref.shape[1]

    def _build_s():
        idx = idx_ref[...]
        u_iota = jax.lax.broadcasted_iota(jnp.int32, (n_units, br), 0)
        s = jnp.zeros((n_units, br), jnp.float32)
        for k in range(idx_ref.shape[0]):
            s = s + jnp.where(u_iota == idx[k : k + 1, :], 1.0, 0.0)
        return s.astype(jnp.bfloat16)

    def _main(sb):
        xf = jnp.dot(
            ph_scr[...], sb, preferred_element_type=jnp.float32
        ) + jnp.dot(plo_scr[...], sb, preferred_element_type=jnp.float32)
        x = xf[:bb, :]
        er = xf[bb : bb + 1, :]
        sd = jnp.dot(
            dh_scr[...], sb, preferred_element_type=jnp.float32
        ) + jnp.dot(dlo_scr[...], sb, preferred_element_type=jnp.float32)

        L = jnp.exp(-er)
        p_min = 1.0 / (1.0 + L)
        p_c = 1.0 / (1.0 + L * jnp.exp(x))
        p_max = 1.0 / (1.0 + L * jnp.exp(sd))
        denom = p_max - p_min
        norm = (p_c - p_min) / (denom + 1e-8)
        norm = jnp.where(denom > 1e-6, norm, 0.0)
        out_ref[...] = jnp.clip(norm, 0.0, 1.0)

    @pl.when(ir == 0)
    def _():
        sb = _build_s()
        c = c_ref[...]
        eo = eo_ref[...]
        ec = ec_ref[...]
        p = jnp.log1p(c * jnp.exp(-ec)) - jnp.log1p(c * jnp.exp(-eo))
        ph, plo = _split_hi_lo(p)
        eh, elo = _split_hi_lo(eps_ref[...])
        zpad = jnp.zeros((7, n_units), jnp.bfloat16)
        ph_scr[...] = jnp.concatenate([ph, eh, zpad], axis=0)
        plo_scr[...] = jnp.concatenate([plo, elo, zpad], axis=0)
        dh_scr[...], dlo_scr[...] = _split_hi_lo(eo - ec)
        _main(sb)

    @pl.when(ir != 0)
    def _():
        _main(_build_s())


@jax.jit
def kernel(energies, concentrations, receptor_indices, epsilon_units):
    b, u, _ = energies.shape
    r, k = receptor_indices.shape
    br = 512
    nr = r // br

    one = jnp.float32(1.0)
    eo = energies[:, :, 0] * one
    ec = energies[:, :, 1] * one
    c2 = concentrations.reshape(b, 1)
    idxt = receptor_indices.T
    eps2 = epsilon_units.reshape(1, u)

    return pl.pallas_call(
        _mwc_kernel,
        grid=(nr,),
        in_specs=[
            pl.BlockSpec((b, u), lambda ir: (0, 0)),
            pl.BlockSpec((b, u), lambda ir: (0, 0)),
            pl.BlockSpec((b, 1), lambda ir: (0, 0)),
            pl.BlockSpec((k, br), lambda ir: (0, ir)),
            pl.BlockSpec((1, u), lambda ir: (0, 0)),
        ],
        out_specs=pl.BlockSpec((b, br), lambda ir: (0, ir)),
        out_shape=jax.ShapeDtypeStruct((b, r), jnp.float32),
        scratch_shapes=[
            pltpu.VMEM((b + 8, u), jnp.bfloat16),
            pltpu.VMEM((b + 8, u), jnp.bfloat16),
            pltpu.VMEM((b, u), jnp.bfloat16),
            pltpu.VMEM((b, u), jnp.bfloat16),
        ],
    )(eo, ec, c2, idxt, eps2)


# speedup vs baseline: 1.2481x; 1.0008x over previous
import jax
import jax.numpy as jnp
from jax.experimental import pallas as pl
from jax.experimental.pallas import tpu as pltpu


def _split_hi_lo(v):
    hi = v.astype(jnp.bfloat16)
    lo = (v - hi.astype(jnp.float32)).astype(jnp.bfloat16)
    return hi, lo


def _mwc_kernel(
    eo_ref, ec_ref, c_ref, idx_ref, eps_ref, out_ref,
    ph_scr, plo_scr, dh_scr, dlo_scr,
):
    ir = pl.program_id(0)
    n_units = eo_ref.shape[1]
    bb = out_ref.shape[0]
    br = out_ref.shape[1]

    def _build_s():
        idx = idx_ref[...]
        u_iota = jax.lax.broadcasted_iota(jnp.int32, (n_units, br), 0)
        s = jnp.zeros((n_units, br), jnp.float32)
        for k in range(idx_ref.shape[0]):
            s = s + jnp.where(u_iota == idx[k : k + 1, :], 1.0, 0.0)
        return s.astype(jnp.bfloat16)

    def _main(sb):
        xf = jnp.dot(
            ph_scr[...], sb, preferred_element_type=jnp.float32
        ) + jnp.dot(plo_scr[...], sb, preferred_element_type=jnp.float32)
        x = xf[:bb, :]
        er = xf[bb : bb + 1, :]
        sd = jnp.dot(
            dh_scr[...], sb, preferred_element_type=jnp.float32
        ) + jnp.dot(dlo_scr[...], sb, preferred_element_type=jnp.float32)

        L = jnp.exp(-er)
        p_min = 1.0 / (1.0 + L)
        p_c = 1.0 / (1.0 + L * jnp.exp(x))
        p_max = 1.0 / (1.0 + L * jnp.exp(sd))
        denom = p_max - p_min
        norm = (p_c - p_min) / (denom + 1e-8)
        norm = jnp.where(denom > 1e-6, norm, 0.0)
        out_ref[...] = jnp.clip(norm, 0.0, 1.0)

    @pl.when(ir == 0)
    def _():
        sb = _build_s()
        c = c_ref[...]
        eo = eo_ref[...]
        ec = ec_ref[...]
        p = jnp.log1p(c * jnp.exp(-ec)) - jnp.log1p(c * jnp.exp(-eo))
        ph, plo = _split_hi_lo(p)
        eh, elo = _split_hi_lo(eps_ref[...])
        ph_scr[0:bb, :] = ph
        ph_scr[bb : bb + 1, :] = eh
        plo_scr[0:bb, :] = plo
        plo_scr[bb : bb + 1, :] = elo
        dh_scr[...], dlo_scr[...] = _split_hi_lo(eo - ec)
        _main(sb)

    @pl.when(ir != 0)
    def _():
        _main(_build_s())


@jax.jit
def kernel(energies, concentrations, receptor_indices, epsilon_units):
    b, u, _ = energies.shape
    r, k = receptor_indices.shape
    br = 512
    nr = r // br

    one = jnp.float32(1.0)
    eo = energies[:, :, 0] * one
    ec = energies[:, :, 1] * one
    c2 = concentrations.reshape(b, 1)
    idxt = receptor_indices.T
    eps2 = epsilon_units.reshape(1, u)

    return pl.pallas_call(
        _mwc_kernel,
        grid=(nr,),
        in_specs=[
            pl.BlockSpec((b, u), lambda ir: (0, 0)),
            pl.BlockSpec((b, u), lambda ir: (0, 0)),
            pl.BlockSpec((b, 1), lambda ir: (0, 0)),
            pl.BlockSpec((k, br), lambda ir: (0, ir)),
            pl.BlockSpec((1, u), lambda ir: (0, 0)),
        ],
        out_specs=pl.BlockSpec((b, br), lambda ir: (0, ir)),
        out_shape=jax.ShapeDtypeStruct((b, r), jnp.float32),
        scratch_shapes=[
            pltpu.VMEM((b + 8, u), jnp.bfloat16),
            pltpu.VMEM((b + 8, u), jnp.bfloat16),
            pltpu.VMEM((b, u), jnp.bfloat16),
            pltpu.VMEM((b, u), jnp.bfloat16),
        ],
    )(eo, ec, c2, idxt, eps2)
